# baseline (speedup 1.0000x reference)
import functools

import jax
import jax.numpy as jnp
from jax import lax
from jax.experimental import pallas as pl
from jax.experimental.pallas import tpu as pltpu
from jax.experimental.pallas import tpu_sc as plsc

N = 10000
E = 160000
IN_DIM = 256
HID_DIM = 128
OUT_DIM = 64

NPAD = 10240
EPAD = 163840
NC, NS = 2, 16
NW = NC * NS
EPT = EPAD // NW
CK = 128
NCH = EPT // CK
EROWS = EPAD // CK
RPT = NPAD // NS
ZR = 64
NB = 4

DEGW = 16

@functools.lru_cache(maxsize=None)
def _sc_mesh():
    return plsc.VectorSubcoreMesh(core_axis_name="c", subcore_axis_name="s")


def _zero_vmem(buf, rows, width):
    z = jnp.zeros((16,), jnp.float32)
    for i in range(rows):
        for j in range(width // 16):
            buf[i, pl.ds(j * 16, 16)] = z


@functools.lru_cache(maxsize=None)
def _make_deg_kernel():
    return functools.partial(
        pl.kernel,
        mesh=_sc_mesh(),
        compiler_params=pltpu.CompilerParams(use_tc_tiling_on_sc=False),
        out_type=jax.ShapeDtypeStruct((NC * NPAD, DEGW), jnp.float32),
        scratch_types=[
            pltpu.VMEM((NCH, CK), jnp.int32),
            pltpu.VMEM((CK, DEGW), jnp.float32),
            pltpu.VMEM((ZR, DEGW), jnp.float32),
            pltpu.VMEM_SHARED((NPAD, DEGW), jnp.float32),
            pltpu.SemaphoreType.DMA,
        ],
    )(_deg_body)


def _deg_body(dst_hbm, out_hbm, didx, ones, zbuf, acc, sem):
    c = lax.axis_index("c")
    s = lax.axis_index("s")
    wid = c * NS + s
    _zero_vmem(zbuf, ZR, DEGW)
    e0 = jnp.where(lax.iota(jnp.int32, 16) == 0, 1.0, 0.0).astype(jnp.float32)
    for i in range(CK):
        ones[i, pl.ds(0, 16)] = e0
    pltpu.sync_copy(dst_hbm.at[pl.ds(wid * NCH, NCH)], didx)
    rbase = s * RPT
    for k in range(RPT // ZR):
        pltpu.sync_copy(zbuf, acc.at[pl.ds(rbase + k * ZR, ZR)])
    plsc.subcore_barrier()
    descs = [
        pltpu.async_copy(ones, acc.at[didx.at[i]], sem, add=True)
        for i in range(NCH)
    ]
    for d in descs:
        d.wait()
    plsc.subcore_barrier()
    obase = c * NPAD + rbase
    pltpu.sync_copy(acc.at[pl.ds(rbase, RPT)], out_hbm.at[pl.ds(obase, RPT)])


CH0 = 80
SEGCH = 40


@functools.lru_cache(maxsize=None)
def _make_agg_kernel(D):
    nb = 2 if D > 64 else 4

    @functools.partial(
        pl.kernel,
        mesh=_sc_mesh(),
        compiler_params=pltpu.CompilerParams(use_tc_tiling_on_sc=False),
        out_type=jax.ShapeDtypeStruct((NPAD, D), jnp.float32),
        scratch_types=[
            pltpu.VMEM((SEGCH, CK), jnp.int32),
            pltpu.VMEM((SEGCH, CK), jnp.int32),
            [pltpu.VMEM((CK, D), jnp.float32)] * nb,
            pltpu.VMEM_SHARED((NPAD, D), jnp.float32),
            [pltpu.SemaphoreType.DMA] * nb,
            [pltpu.SemaphoreType.DMA] * nb,
        ],
    )
    def agg(y_hbm, src_hbm, dst_hbm, out_hbm, sidx, didx, rows, acc,
            gsem, ssem):
        NB = nb
        c = lax.axis_index("c")
        s = lax.axis_index("s")

        def run_seg(seg, cbase):
            pltpu.sync_copy(src_hbm.at[pl.ds(cbase, seg)],
                            sidx.at[pl.ds(0, seg)])
            pltpu.sync_copy(dst_hbm.at[pl.ds(cbase, seg)],
                            didx.at[pl.ds(0, seg)])
            g = [None] * NB
            sc = [None] * NB
            for i in range(min(NB, seg)):
                g[i] = pltpu.async_copy(y_hbm.at[sidx.at[i]], rows[i], gsem[i])
            for i in range(seg):
                b = i % NB
                g[b].wait()
                sc[b] = pltpu.async_copy(rows[b], acc.at[didx.at[i]], ssem[b],
                                         add=True)
                j = i + NB
                if j < seg:
                    sc[b].wait()
                    g[b] = pltpu.async_copy(y_hbm.at[sidx.at[j]], rows[b],
                                            gsem[b])
            for i in range(max(0, seg - NB), seg):
                sc[i % NB].wait()

        def run_chunks(nch, cbase):
            for s0 in range(0, nch, SEGCH):
                run_seg(min(SEGCH, nch - s0), cbase + s0)

        @pl.when(c == 0)
        def _():
            rbase = s * RPT
            _zero_vmem(rows[0], CK, D)
            for k in range(RPT // CK):
                pltpu.sync_copy(rows[0], acc.at[pl.ds(rbase + k * CK, CK)])
            plsc.subcore_barrier()
            run_chunks(CH0, s * CH0)
            plsc.subcore_barrier()
            pltpu.sync_copy(acc.at[pl.ds(rbase, RPT)],
                            out_hbm.at[pl.ds(rbase, RPT)])

    return agg


_BR = 2048
_G = NPAD // _BR


def _dis_block(d0, d1):
    deg = d0[:, 0:1] + d1[:, 0:1] + 1.0
    return lax.rsqrt(deg)


def _tc1_body(x_ref, w1_ref, d0_ref, d1_ref, y1_ref):
    dis = _dis_block(d0_ref[...], d1_ref[...])
    xw = lax.dot_general(
        x_ref[...], w1_ref[...], (((1,), (0,)), ((), ())),
        precision=lax.Precision.HIGHEST, preferred_element_type=jnp.float32)
    y1_ref[...] = dis * xw


def _tc2_body(p0_ref, y1_ref, d0_ref, d1_ref, b1_ref, w2_ref, y2_ref):
    dis = _dis_block(d0_ref[...], d1_ref[...])
    h = jnp.maximum(dis * (p0_ref[...] + y1_ref[...]) + b1_ref[...], 0.0)
    hw = lax.dot_general(
        h, w2_ref[...], (((1,), (0,)), ((), ())),
        precision=lax.Precision.HIGHEST, preferred_element_type=jnp.float32)
    y2_ref[...] = dis * hw


def _tc3_body(q0_ref, y2_ref, d0_ref, d1_ref, b2_ref, o_ref):
    dis = _dis_block(d0_ref[...], d1_ref[...])
    o_ref[...] = dis * (q0_ref[...] + y2_ref[...]) + b2_ref[...]


def _rows(i):
    return (i, 0)


def _rows_hi(i):
    return (i + _G, 0)


def _whole(i):
    return (0, 0)


_tc1 = pl.pallas_call(
    _tc1_body,
    grid=(_G,),
    in_specs=[
        pl.BlockSpec((_BR, IN_DIM), _rows),
        pl.BlockSpec((IN_DIM, HID_DIM), _whole),
        pl.BlockSpec((_BR, DEGW), _rows),
        pl.BlockSpec((_BR, DEGW), _rows_hi),
    ],
    out_specs=pl.BlockSpec((_BR, HID_DIM), _rows),
    out_shape=jax.ShapeDtypeStruct((NPAD, HID_DIM), jnp.float32),
)

_tc2 = pl.pallas_call(
    _tc2_body,
    grid=(_G,),
    in_specs=[
        pl.BlockSpec((_BR, HID_DIM), _rows),
        pl.BlockSpec((_BR, HID_DIM), _rows),
        pl.BlockSpec((_BR, DEGW), _rows),
        pl.BlockSpec((_BR, DEGW), _rows_hi),
        pl.BlockSpec((1, HID_DIM), _whole),
        pl.BlockSpec((HID_DIM, OUT_DIM), _whole),
    ],
    out_specs=pl.BlockSpec((_BR, OUT_DIM), _rows),
    out_shape=jax.ShapeDtypeStruct((NPAD, OUT_DIM), jnp.float32),
)

_tc3 = pl.pallas_call(
    _tc3_body,
    grid=(_G,),
    in_specs=[
        pl.BlockSpec((_BR, OUT_DIM), _rows),
        pl.BlockSpec((_BR, OUT_DIM), _rows),
        pl.BlockSpec((_BR, DEGW), _rows),
        pl.BlockSpec((_BR, DEGW), _rows_hi),
        pl.BlockSpec((1, OUT_DIM), _whole),
    ],
    out_specs=pl.BlockSpec((_BR, OUT_DIM), _rows),
    out_shape=jax.ShapeDtypeStruct((NPAD, OUT_DIM), jnp.float32),
)


@jax.jit
def kernel(x, edge_index, W1, b1, W2, b2):
    src = edge_index[0].astype(jnp.int32)
    dst = edge_index[1].astype(jnp.int32)
    pad = jnp.full((EPAD - E,), N, dtype=jnp.int32)
    src2 = jnp.concatenate([src, pad]).reshape(EROWS, CK)
    dst2 = jnp.concatenate([dst, pad]).reshape(EROWS, CK)
    xp = jnp.pad(x, ((0, NPAD - N), (0, 0)))

    degp = _make_deg_kernel()(dst2)
    y1 = _tc1(xp, W1, degp, degp)
    p = _make_agg_kernel(HID_DIM)(y1, src2, dst2)
    y2 = _tc2(p, y1, degp, degp, b1.reshape(1, HID_DIM), W2)
    q = _make_agg_kernel(OUT_DIM)(y2, src2, dst2)
    out = _tc3(q, y2, degp, degp, b2.reshape(1, OUT_DIM))
    return out[:N]


# speedup vs baseline: 1.9006x; 1.9006x over previous
import functools

import jax
import jax.numpy as jnp
from jax import lax
from jax.experimental import pallas as pl
from jax.experimental.pallas import tpu as pltpu
from jax.experimental.pallas import tpu_sc as plsc

N = 10000
E = 160000
IN_DIM = 256
HID_DIM = 128
OUT_DIM = 64

NPAD = 10240
EPAD = 163840
NC, NS = 2, 16
NW = NC * NS
EPT = EPAD // NW
CK = 128
NCH = EPT // CK
EROWS = EPAD // CK
RPT = NPAD // NS
ZR = 64
NB = 4

DEGW = 16

@functools.lru_cache(maxsize=None)
def _sc_mesh():
    return plsc.VectorSubcoreMesh(core_axis_name="c", subcore_axis_name="s")


def _zero_vmem(buf, rows, width):
    z = jnp.zeros((16,), jnp.float32)
    for i in range(rows):
        for j in range(width // 16):
            buf[i, pl.ds(j * 16, 16)] = z


@functools.lru_cache(maxsize=None)
def _make_deg_kernel():
    return functools.partial(
        pl.kernel,
        mesh=_sc_mesh(),
        compiler_params=pltpu.CompilerParams(use_tc_tiling_on_sc=False),
        out_type=jax.ShapeDtypeStruct((NC * NPAD, DEGW), jnp.float32),
        scratch_types=[
            pltpu.VMEM((NCH, CK), jnp.int32),
            pltpu.VMEM((CK, DEGW), jnp.float32),
            pltpu.VMEM((ZR, DEGW), jnp.float32),
            pltpu.VMEM_SHARED((NPAD, DEGW), jnp.float32),
            pltpu.SemaphoreType.DMA,
        ],
    )(_deg_body)


def _deg_body(dst_hbm, out_hbm, didx, ones, zbuf, acc, sem):
    c = lax.axis_index("c")
    s = lax.axis_index("s")
    wid = c * NS + s
    _zero_vmem(zbuf, ZR, DEGW)
    e0 = jnp.where(lax.iota(jnp.int32, 16) == 0, 1.0, 0.0).astype(jnp.float32)
    for i in range(CK):
        ones[i, pl.ds(0, 16)] = e0
    pltpu.sync_copy(dst_hbm.at[pl.ds(wid * NCH, NCH)], didx)
    rbase = s * RPT
    for k in range(RPT // ZR):
        pltpu.sync_copy(zbuf, acc.at[pl.ds(rbase + k * ZR, ZR)])
    plsc.subcore_barrier()
    descs = [
        pltpu.async_copy(ones, acc.at[didx.at[i]], sem, add=True)
        for i in range(NCH)
    ]
    for d in descs:
        d.wait()
    plsc.subcore_barrier()
    obase = c * NPAD + rbase
    pltpu.sync_copy(acc.at[pl.ds(rbase, RPT)], out_hbm.at[pl.ds(obase, RPT)])


CH0 = 80
SEGCH = 40


@functools.lru_cache(maxsize=None)
def _make_agg_kernel(D):
    nb = 2 if D > 64 else 4

    @functools.partial(
        pl.kernel,
        mesh=_sc_mesh(),
        compiler_params=pltpu.CompilerParams(use_tc_tiling_on_sc=False),
        out_type=jax.ShapeDtypeStruct((NPAD, D), jnp.float32),
        scratch_types=[
            pltpu.VMEM((SEGCH, CK), jnp.int32),
            pltpu.VMEM((SEGCH, CK), jnp.int32),
            [pltpu.VMEM((CK, D), jnp.float32)] * nb,
            pltpu.VMEM_SHARED((NPAD, D), jnp.float32),
            [pltpu.SemaphoreType.DMA] * nb,
            [pltpu.SemaphoreType.DMA] * nb,
        ],
    )
    def agg(y_hbm, src_hbm, dst_hbm, out_hbm, sidx, didx, rows, acc,
            gsem, ssem):
        NB = nb
        c = lax.axis_index("c")
        s = lax.axis_index("s")

        def run_seg(seg, cbase):
            pltpu.sync_copy(src_hbm.at[pl.ds(cbase, seg)],
                            sidx.at[pl.ds(0, seg)])
            pltpu.sync_copy(dst_hbm.at[pl.ds(cbase, seg)],
                            didx.at[pl.ds(0, seg)])
            g = [None] * NB
            sc = [None] * NB
            for i in range(min(NB, seg)):
                g[i] = pltpu.async_copy(y_hbm.at[sidx.at[i]], rows[i], gsem[i])
            for i in range(seg):
                b = i % NB
                g[b].wait()
                sc[b] = pltpu.async_copy(rows[b], acc.at[didx.at[i]], ssem[b],
                                         add=True)
                j = i + NB
                if j < seg:
                    sc[b].wait()
                    g[b] = pltpu.async_copy(y_hbm.at[sidx.at[j]], rows[b],
                                            gsem[b])
            for i in range(max(0, seg - NB), seg):
                sc[i % NB].wait()

        def run_chunks(nch, cbase):
            for s0 in range(0, nch, SEGCH):
                run_seg(min(SEGCH, nch - s0), cbase + s0)

        @pl.when(c == 0)
        def _():
            rbase = s * RPT
            _zero_vmem(rows[0], CK, D)
            for k in range(RPT // CK):
                pltpu.sync_copy(rows[0], acc.at[pl.ds(rbase + k * CK, CK)])
            plsc.subcore_barrier()
            run_chunks(CH0, s * CH0)
            plsc.subcore_barrier()
            pltpu.sync_copy(acc.at[pl.ds(rbase, RPT)],
                            out_hbm.at[pl.ds(rbase, RPT)])

    return agg


_BR = 2048
_G = NPAD // _BR


def _dis_block(d0, d1):
    deg = d0[:, 0:1] + d1[:, 0:1] + 1.0
    return lax.rsqrt(deg)


def _tc1_body(x_ref, w1_ref, d0_ref, d1_ref, y1_ref):
    dis = _dis_block(d0_ref[...], d1_ref[...])
    xw = lax.dot_general(
        x_ref[...], w1_ref[...], (((1,), (0,)), ((), ())),
        precision=lax.Precision.HIGHEST, preferred_element_type=jnp.float32)
    y1_ref[...] = dis * xw


def _tc2_body(p0_ref, y1_ref, d0_ref, d1_ref, b1_ref, w2_ref, y2_ref):
    dis = _dis_block(d0_ref[...], d1_ref[...])
    h = jnp.maximum(dis * (p0_ref[...] + y1_ref[...]) + b1_ref[...], 0.0)
    hw = lax.dot_general(
        h, w2_ref[...], (((1,), (0,)), ((), ())),
        precision=lax.Precision.HIGHEST, preferred_element_type=jnp.float32)
    y2_ref[...] = dis * hw


def _tc3_body(q0_ref, y2_ref, d0_ref, d1_ref, b2_ref, o_ref):
    dis = _dis_block(d0_ref[...], d1_ref[...])
    o_ref[...] = dis * (q0_ref[...] + y2_ref[...]) + b2_ref[...]


def _rows(i):
    return (i, 0)


def _rows_hi(i):
    return (i + _G, 0)


def _whole(i):
    return (0, 0)


_tc1 = pl.pallas_call(
    _tc1_body,
    grid=(_G,),
    in_specs=[
        pl.BlockSpec((_BR, IN_DIM), _rows),
        pl.BlockSpec((IN_DIM, HID_DIM), _whole),
        pl.BlockSpec((_BR, DEGW), _rows),
        pl.BlockSpec((_BR, DEGW), _rows_hi),
    ],
    out_specs=pl.BlockSpec((_BR, HID_DIM), _rows),
    out_shape=jax.ShapeDtypeStruct((NPAD, HID_DIM), jnp.float32),
)

_tc2 = pl.pallas_call(
    _tc2_body,
    grid=(_G,),
    in_specs=[
        pl.BlockSpec((_BR, HID_DIM), _rows),
        pl.BlockSpec((_BR, HID_DIM), _rows),
        pl.BlockSpec((_BR, DEGW), _rows),
        pl.BlockSpec((_BR, DEGW), _rows_hi),
        pl.BlockSpec((1, HID_DIM), _whole),
        pl.BlockSpec((HID_DIM, OUT_DIM), _whole),
    ],
    out_specs=pl.BlockSpec((_BR, OUT_DIM), _rows),
    out_shape=jax.ShapeDtypeStruct((NPAD, OUT_DIM), jnp.float32),
)

_tc3 = pl.pallas_call(
    _tc3_body,
    grid=(_G,),
    in_specs=[
        pl.BlockSpec((_BR, OUT_DIM), _rows),
        pl.BlockSpec((_BR, OUT_DIM), _rows),
        pl.BlockSpec((_BR, DEGW), _rows),
        pl.BlockSpec((_BR, DEGW), _rows_hi),
        pl.BlockSpec((1, OUT_DIM), _whole),
    ],
    out_specs=pl.BlockSpec((_BR, OUT_DIM), _rows),
    out_shape=jax.ShapeDtypeStruct((NPAD, OUT_DIM), jnp.float32),
)


@jax.jit
def kernel(x, edge_index, W1, b1, W2, b2):
    src = edge_index[0].astype(jnp.int32)
    dst = edge_index[1].astype(jnp.int32)
    pad = N + (jnp.arange(EPAD - E, dtype=jnp.int32) % (NPAD - N))
    src2 = jnp.concatenate([src, pad]).reshape(EROWS, CK)
    dst2 = jnp.concatenate([dst, pad]).reshape(EROWS, CK)
    xp = jnp.pad(x, ((0, NPAD - N), (0, 0)))

    degp = _make_deg_kernel()(dst2)
    y1 = _tc1(xp, W1, degp, degp)
    p = _make_agg_kernel(HID_DIM)(y1, src2, dst2)
    y2 = _tc2(p, y1, degp, degp, b1.reshape(1, HID_DIM), W2)
    q = _make_agg_kernel(OUT_DIM)(y2, src2, dst2)
    out = _tc3(q, y2, degp, degp, b2.reshape(1, OUT_DIM))
    return out[:N]


# speedup vs baseline: 2.3431x; 1.2328x over previous
import functools

import jax
import jax.numpy as jnp
from jax import lax
from jax.experimental import pallas as pl
from jax.experimental.pallas import tpu as pltpu
from jax.experimental.pallas import tpu_sc as plsc

N = 10000
E = 160000
IN_DIM = 256
HID_DIM = 128
OUT_DIM = 64

NPAD = 10240
EPAD = 163840
NC, NS = 2, 16
NW = NC * NS
EPT = EPAD // NW
CK = 128
NCH = EPT // CK
EROWS = EPAD // CK
RPT = NPAD // NS
ZR = 64
NB = 4

DEGW = 16

@functools.lru_cache(maxsize=None)
def _sc_mesh():
    return plsc.VectorSubcoreMesh(core_axis_name="c", subcore_axis_name="s")


def _zero_vmem(buf, rows, width):
    z = jnp.zeros((16,), jnp.float32)
    for i in range(rows):
        for j in range(width // 16):
            buf[i, pl.ds(j * 16, 16)] = z


@functools.lru_cache(maxsize=None)
def _make_deg_kernel():
    return functools.partial(
        pl.kernel,
        mesh=_sc_mesh(),
        compiler_params=pltpu.CompilerParams(use_tc_tiling_on_sc=False),
        out_type=jax.ShapeDtypeStruct((NC * NPAD, DEGW), jnp.float32),
        scratch_types=[
            pltpu.VMEM((NCH, CK), jnp.int32),
            pltpu.VMEM((CK, DEGW), jnp.float32),
            pltpu.VMEM((ZR, DEGW), jnp.float32),
            pltpu.VMEM_SHARED((NPAD, DEGW), jnp.float32),
            pltpu.SemaphoreType.DMA,
        ],
    )(_deg_body)


def _deg_body(dst_hbm, out_hbm, didx, ones, zbuf, acc, sem):
    c = lax.axis_index("c")
    s = lax.axis_index("s")
    wid = c * NS + s
    _zero_vmem(zbuf, ZR, DEGW)
    e0 = jnp.where(lax.iota(jnp.int32, 16) == 0, 1.0, 0.0).astype(jnp.float32)
    for i in range(CK):
        ones[i, pl.ds(0, 16)] = e0
    pltpu.sync_copy(dst_hbm.at[pl.ds(wid * NCH, NCH)], didx)
    rbase = s * RPT
    for k in range(RPT // ZR):
        pltpu.sync_copy(zbuf, acc.at[pl.ds(rbase + k * ZR, ZR)])
    plsc.subcore_barrier()
    descs = [
        pltpu.async_copy(ones, acc.at[didx.at[i]], sem, add=True)
        for i in range(NCH)
    ]
    for d in descs:
        d.wait()
    plsc.subcore_barrier()
    obase = c * NPAD + rbase
    pltpu.sync_copy(acc.at[pl.ds(rbase, RPT)], out_hbm.at[pl.ds(obase, RPT)])


CH0 = 40
CH1 = 40
SEGCH = 40


@functools.lru_cache(maxsize=None)
def _make_agg_kernel(D):
    nb = 2 if D > 64 else 4

    @functools.partial(
        pl.kernel,
        mesh=_sc_mesh(),
        compiler_params=pltpu.CompilerParams(use_tc_tiling_on_sc=False),
        out_type=jax.ShapeDtypeStruct((NC * NPAD, D), jnp.float32),
        scratch_types=[
            pltpu.VMEM((SEGCH, CK), jnp.int32),
            pltpu.VMEM((SEGCH, CK), jnp.int32),
            [pltpu.VMEM((CK, D), jnp.float32)] * nb,
            pltpu.VMEM_SHARED((NPAD, D), jnp.float32),
            [pltpu.SemaphoreType.DMA] * nb,
            [pltpu.SemaphoreType.DMA] * nb,
        ],
    )
    def agg(y_hbm, src_hbm, dst_hbm, out_hbm, sidx, didx, rows, acc,
            gsem, ssem):
        NB = nb
        c = lax.axis_index("c")
        s = lax.axis_index("s")

        def run_seg(seg, cbase):
            pltpu.sync_copy(src_hbm.at[pl.ds(cbase, seg)],
                            sidx.at[pl.ds(0, seg)])
            pltpu.sync_copy(dst_hbm.at[pl.ds(cbase, seg)],
                            didx.at[pl.ds(0, seg)])
            g = [None] * NB
            sc = [None] * NB
            for i in range(min(NB, seg)):
                g[i] = pltpu.async_copy(y_hbm.at[sidx.at[i]], rows[i], gsem[i])
            for i in range(seg):
                b = i % NB
                g[b].wait()
                sc[b] = pltpu.async_copy(rows[b], acc.at[didx.at[i]], ssem[b],
                                         add=True)
                j = i + NB
                if j < seg:
                    sc[b].wait()
                    g[b] = pltpu.async_copy(y_hbm.at[sidx.at[j]], rows[b],
                                            gsem[b])
            for i in range(max(0, seg - NB), seg):
                sc[i % NB].wait()

        def run_chunks(nch, cbase):
            for s0 in range(0, nch, SEGCH):
                run_seg(min(SEGCH, nch - s0), cbase + s0)

        rbase = s * RPT
        _zero_vmem(rows[0], CK, D)
        for k in range(RPT // CK):
            pltpu.sync_copy(rows[0], acc.at[pl.ds(rbase + k * CK, CK)])
        plsc.subcore_barrier()

        @pl.when(c == 0)
        def _():
            run_chunks(CH0, s * CH0)

        @pl.when(c == 1)
        def _():
            run_chunks(CH1, NS * CH0 + s * CH1)

        plsc.subcore_barrier()
        obase = c * NPAD + rbase
        pltpu.sync_copy(acc.at[pl.ds(rbase, RPT)],
                        out_hbm.at[pl.ds(obase, RPT)])

    return agg


_BR = 2048
_G = NPAD // _BR


def _dis_block(d0, d1):
    deg = d0[:, 0:1] + d1[:, 0:1] + 1.0
    return lax.rsqrt(deg)


def _tc1_body(x_ref, w1_ref, d0_ref, d1_ref, y1_ref):
    dis = _dis_block(d0_ref[...], d1_ref[...])
    xw = lax.dot_general(
        x_ref[...], w1_ref[...], (((1,), (0,)), ((), ())),
        precision=lax.Precision.HIGHEST, preferred_element_type=jnp.float32)
    y1_ref[...] = dis * xw


def _tc2_body(p0_ref, p1_ref, y1_ref, d0_ref, d1_ref, b1_ref, w2_ref, y2_ref):
    dis = _dis_block(d0_ref[...], d1_ref[...])
    h = jnp.maximum(dis * (p0_ref[...] + p1_ref[...] + y1_ref[...]) + b1_ref[...], 0.0)
    hw = lax.dot_general(
        h, w2_ref[...], (((1,), (0,)), ((), ())),
        precision=lax.Precision.HIGHEST, preferred_element_type=jnp.float32)
    y2_ref[...] = dis * hw


def _tc3_body(q0_ref, q1_ref, y2_ref, d0_ref, d1_ref, b2_ref, o_ref):
    dis = _dis_block(d0_ref[...], d1_ref[...])
    o_ref[...] = dis * (q0_ref[...] + q1_ref[...] + y2_ref[...]) + b2_ref[...]


def _rows(i):
    return (i, 0)


def _rows_hi(i):
    return (i + _G, 0)


def _whole(i):
    return (0, 0)


_tc1 = pl.pallas_call(
    _tc1_body,
    grid=(_G,),
    in_specs=[
        pl.BlockSpec((_BR, IN_DIM), _rows),
        pl.BlockSpec((IN_DIM, HID_DIM), _whole),
        pl.BlockSpec((_BR, DEGW), _rows),
        pl.BlockSpec((_BR, DEGW), _rows_hi),
    ],
    out_specs=pl.BlockSpec((_BR, HID_DIM), _rows),
    out_shape=jax.ShapeDtypeStruct((NPAD, HID_DIM), jnp.float32),
)

_tc2 = pl.pallas_call(
    _tc2_body,
    grid=(_G,),
    in_specs=[
        pl.BlockSpec((_BR, HID_DIM), _rows),
        pl.BlockSpec((_BR, HID_DIM), _rows_hi),
        pl.BlockSpec((_BR, HID_DIM), _rows),
        pl.BlockSpec((_BR, DEGW), _rows),
        pl.BlockSpec((_BR, DEGW), _rows_hi),
        pl.BlockSpec((1, HID_DIM), _whole),
        pl.BlockSpec((HID_DIM, OUT_DIM), _whole),
    ],
    out_specs=pl.BlockSpec((_BR, OUT_DIM), _rows),
    out_shape=jax.ShapeDtypeStruct((NPAD, OUT_DIM), jnp.float32),
)

_tc3 = pl.pallas_call(
    _tc3_body,
    grid=(_G,),
    in_specs=[
        pl.BlockSpec((_BR, OUT_DIM), _rows),
        pl.BlockSpec((_BR, OUT_DIM), _rows_hi),
        pl.BlockSpec((_BR, OUT_DIM), _rows),
        pl.BlockSpec((_BR, DEGW), _rows),
        pl.BlockSpec((_BR, DEGW), _rows_hi),
        pl.BlockSpec((1, OUT_DIM), _whole),
    ],
    out_specs=pl.BlockSpec((_BR, OUT_DIM), _rows),
    out_shape=jax.ShapeDtypeStruct((NPAD, OUT_DIM), jnp.float32),
)


@jax.jit
def kernel(x, edge_index, W1, b1, W2, b2):
    src = edge_index[0].astype(jnp.int32)
    dst = edge_index[1].astype(jnp.int32)
    pad = N + (jnp.arange(EPAD - E, dtype=jnp.int32) % (NPAD - N))
    src2 = jnp.concatenate([src, pad]).reshape(EROWS, CK)
    dst2 = jnp.concatenate([dst, pad]).reshape(EROWS, CK)
    xp = jnp.pad(x, ((0, NPAD - N), (0, 0)))

    degp = _make_deg_kernel()(dst2)
    y1 = _tc1(xp, W1, degp, degp)
    p = _make_agg_kernel(HID_DIM)(y1, src2, dst2)
    y2 = _tc2(p, p, y1, degp, degp, b1.reshape(1, HID_DIM), W2)
    q = _make_agg_kernel(OUT_DIM)(y2, src2, dst2)
    out = _tc3(q, q, y2, degp, degp, b2.reshape(1, OUT_DIM))
    return out[:N]
